# baseline (speedup 1.0000x reference)
import functools

import jax
import jax.numpy as jnp
from jax import lax
from jax.experimental import pallas as pl
from jax.experimental.pallas import tpu as pltpu
from jax.experimental.pallas import tpu_sc as plsc

L = 200
B = 10
BP = 16
H = 2
D = 64
N = L * BP
NG = 2048
V8 = 125000


def _sc_gather(ttab, xt):
    info = plsc.get_sparse_core_info()
    nw = info.num_cores * info.num_subcores
    b_per_w = NG // nw
    ck = 8
    mesh = plsc.VectorSubcoreMesh(core_axis_name="c", subcore_axis_name="s")

    nslots = 14

    @functools.partial(
        pl.kernel,
        mesh=mesh,
        out_type=jax.ShapeDtypeStruct((NG, D), jnp.float32),
        compiler_params=pltpu.CompilerParams(needs_layout_passes=False),
        scratch_types=[
            pltpu.VMEM((B, L), jnp.int32),
            pltpu.VMEM((b_per_w,), jnp.int32),
            pltpu.VMEM((D, nslots * 128), jnp.float32),
            pltpu.VMEM((b_per_w, D), jnp.float32),
        ] + [pltpu.SemaphoreType.DMA] * nslots,
    )
    def gather_kernel(ttab_hbm, xt_hbm, out_hbm, xt_v, idx_v, blocks_v,
                      out_v, *sems):
        wid = lax.axis_index("s") * info.num_cores + lax.axis_index("c")
        base = wid * b_per_w
        dvs = [jnp.arange(16, dtype=jnp.int32) + 16 * m
               for m in range(D // 16)]
        pltpu.sync_copy(xt_hbm, xt_v)
        lane = jnp.arange(16, dtype=jnp.int32)
        svals, cvals = [], []
        for jb in range(0, b_per_w, 16):
            n = base + jb + lane
            t = lax.shift_right_logical(n * 6554, 16)
            bb = n - t * 10
            t = jnp.minimum(t, L - 1)
            iv = plsc.load_gather(xt_v, [bb, t])
            for k in range(16):
                e = iv[k]
                svals.append(pl.multiple_of(e & ~jnp.int32(127), 128))
                cvals.append(e & 127)

        def issue(i, slot):
            return pltpu.async_copy(
                ttab_hbm.at[:, pl.ds(svals[i], 128)],
                blocks_v.at[:, pl.ds(128 * slot, 128)], sems[slot])

        def extract(j, slot):
            cf = lax.broadcast(cvals[j] + 128 * slot, (16,))
            jf = jnp.full((16,), j, jnp.int32)
            for dv in dvs:
                vals = plsc.load_gather(blocks_v, [dv, cf])
                plsc.store_scatter(out_v, [jf, dv], vals)

        descs = [None] * nslots
        for i in range(b_per_w + nslots):
            slot = i % nslots
            if i >= nslots:
                descs[slot].wait()
                extract(i - nslots, slot)
            if i < b_per_w:
                descs[slot] = issue(i, slot)
        pltpu.sync_copy(out_v, out_hbm.at[pl.ds(base, b_per_w)])

    return gather_kernel(ttab, xt)


def _rnn_body(z_ref, wih0_ref, wih1_ref, whh_ref, bias_ref, fcw0_ref,
              fcw1_ref, fcb_ref, out_ref, a0_ref, a1_ref, ps_ref):
    a0_ref[...] = (jnp.dot(z_ref[...], wih0_ref[...],
                           preferred_element_type=jnp.float32)
                   + bias_ref[0, 0])
    a1_ref[...] = (jnp.dot(z_ref[...], wih1_ref[...],
                           preferred_element_type=jnp.float32)
                   + bias_ref[0, 1])
    w00 = whh_ref[0, 0]
    w01 = whh_ref[0, 1]
    w10 = whh_ref[1, 0]
    w11 = whh_ref[1, 1]
    fcw0 = fcw0_ref[...]
    fcw1 = fcw1_ref[...]

    def step(t, carry):
        h0, h1 = carry
        a0 = a0_ref[pl.ds(t * BP, BP), :]
        a1 = a1_ref[pl.ds(t * BP, BP), :]
        n0 = jnp.tanh(a0 + h0 * w00 + h1 * w01)
        n1 = jnp.tanh(a1 + h0 * w10 + h1 * w11)
        ps_ref[t] = n0 * fcw0 + n1 * fcw1
        return (n0, n1)

    zero = jnp.zeros((BP, 1), jnp.float32)
    lax.fori_loop(0, L, step, (zero, zero))
    s = jnp.sum(jnp.sum(ps_ref[...], axis=2), axis=1)
    out_ref[...] = jax.nn.sigmoid(s + fcb_ref[0, 0])[:, None]


def _rnn_head(z, wih0, wih1, whh, bias, fcw0, fcw1, fcb):
    return pl.pallas_call(
        _rnn_body,
        out_shape=jax.ShapeDtypeStruct((L, 1), jnp.float32),
        in_specs=[
            pl.BlockSpec(memory_space=pltpu.VMEM),
            pl.BlockSpec(memory_space=pltpu.VMEM),
            pl.BlockSpec(memory_space=pltpu.VMEM),
            pl.BlockSpec(memory_space=pltpu.SMEM),
            pl.BlockSpec(memory_space=pltpu.SMEM),
            pl.BlockSpec(memory_space=pltpu.VMEM),
            pl.BlockSpec(memory_space=pltpu.VMEM),
            pl.BlockSpec(memory_space=pltpu.SMEM),
        ],
        out_specs=pl.BlockSpec(memory_space=pltpu.VMEM),
        scratch_shapes=[
            pltpu.VMEM((N, 1), jnp.float32),
            pltpu.VMEM((N, 1), jnp.float32),
            pltpu.VMEM((L, BP, 1), jnp.float32),
        ],
    )(z, wih0, wih1, whh, bias, fcw0, fcw1, fcb)


def kernel(x, emb_table, W_ih, W_hh, b_ih, b_hh, fc_W, fc_b):
    z2 = _sc_gather(emb_table.T, x.astype(jnp.int32).T)
    z = jnp.pad(z2[:L * B].reshape(L, B, D),
                ((0, 0), (0, BP - B), (0, 0))).reshape(N, D)

    wih_t = W_ih.astype(jnp.float32).T
    wih0 = wih_t[:, 0:1]
    wih1 = wih_t[:, 1:2]
    bias = (b_ih + b_hh).astype(jnp.float32).reshape(1, H)
    fcw = fc_W.astype(jnp.float32).reshape(B, H)
    fcw0 = jnp.zeros((BP, 1), jnp.float32).at[:B, 0].set(fcw[:, 0])
    fcw1 = jnp.zeros((BP, 1), jnp.float32).at[:B, 0].set(fcw[:, 1])
    whh = W_hh.astype(jnp.float32)
    fcb = fc_b.astype(jnp.float32).reshape(1, 1)

    return _rnn_head(z, wih0, wih1, whh, bias, fcw0, fcw1, fcb)


# speedup vs baseline: 1.0092x; 1.0092x over previous
import functools

import jax
import jax.numpy as jnp
from jax import lax
from jax.experimental import pallas as pl
from jax.experimental.pallas import tpu as pltpu
from jax.experimental.pallas import tpu_sc as plsc

L = 200
B = 10
BP = 16
H = 2
D = 64
N = L * BP
NG = 2048
V8 = 125000


def _sc_gather(ttab, xt):
    info = plsc.get_sparse_core_info()
    nw = info.num_cores * info.num_subcores
    b_per_w = NG // nw
    ck = 8
    mesh = plsc.VectorSubcoreMesh(core_axis_name="c", subcore_axis_name="s")

    nslots = 12

    @functools.partial(
        pl.kernel,
        mesh=mesh,
        out_type=jax.ShapeDtypeStruct((NG, D), jnp.float32),
        compiler_params=pltpu.CompilerParams(needs_layout_passes=False),
        scratch_types=[
            pltpu.VMEM((B, L), jnp.int32),
            pltpu.VMEM((b_per_w,), jnp.int32),
            pltpu.VMEM((D, nslots * 128), jnp.float32),
            pltpu.VMEM((b_per_w, D), jnp.float32),
        ] + [pltpu.SemaphoreType.DMA] * nslots,
    )
    def gather_kernel(ttab_hbm, xt_hbm, out_hbm, xt_v, idx_v, blocks_v,
                      out_v, *sems):
        wid = lax.axis_index("s") * info.num_cores + lax.axis_index("c")
        base = wid * b_per_w
        dvs = [jnp.arange(16, dtype=jnp.int32) + 16 * m
               for m in range(D // 16)]
        pltpu.sync_copy(xt_hbm, xt_v)
        lane = jnp.arange(16, dtype=jnp.int32)
        svals, cvals = [], []
        for jb in range(0, b_per_w, 16):
            n = base + jb + lane
            t = lax.shift_right_logical(n * 6554, 16)
            bb = n - t * 10
            t = jnp.minimum(t, L - 1)
            iv = plsc.load_gather(xt_v, [bb, t])
            for k in range(16):
                e = iv[k]
                svals.append(pl.multiple_of(e & ~jnp.int32(127), 128))
                cvals.append(e & 127)

        def issue(i, slot):
            return pltpu.async_copy(
                ttab_hbm.at[:, pl.ds(svals[i], 128)],
                blocks_v.at[:, pl.ds(128 * slot, 128)], sems[slot])

        def extract(j, slot):
            cf = lax.broadcast(cvals[j] + 128 * slot, (16,))
            jf = jnp.full((16,), j, jnp.int32)
            for dv in dvs:
                vals = plsc.load_gather(blocks_v, [dv, cf])
                plsc.store_scatter(out_v, [jf, dv], vals)

        descs = [None] * nslots
        for i in range(b_per_w + nslots):
            slot = i % nslots
            if i >= nslots:
                descs[slot].wait()
                extract(i - nslots, slot)
            if i < b_per_w:
                descs[slot] = issue(i, slot)
        pltpu.sync_copy(out_v, out_hbm.at[pl.ds(base, b_per_w)])

    return gather_kernel(ttab, xt)


def _rnn_body(z_ref, wih0_ref, wih1_ref, whh_ref, bias_ref, fcw0_ref,
              fcw1_ref, fcb_ref, out_ref, a0_ref, a1_ref, ps_ref):
    a0_ref[...] = (jnp.dot(z_ref[...], wih0_ref[...],
                           preferred_element_type=jnp.float32)
                   + bias_ref[0, 0])
    a1_ref[...] = (jnp.dot(z_ref[...], wih1_ref[...],
                           preferred_element_type=jnp.float32)
                   + bias_ref[0, 1])
    w00 = whh_ref[0, 0]
    w01 = whh_ref[0, 1]
    w10 = whh_ref[1, 0]
    w11 = whh_ref[1, 1]
    fcw0 = fcw0_ref[...]
    fcw1 = fcw1_ref[...]

    def step(t, carry):
        h0, h1 = carry
        a0 = a0_ref[pl.ds(t * BP, BP), :]
        a1 = a1_ref[pl.ds(t * BP, BP), :]
        n0 = jnp.tanh(a0 + h0 * w00 + h1 * w01)
        n1 = jnp.tanh(a1 + h0 * w10 + h1 * w11)
        ps_ref[t] = n0 * fcw0 + n1 * fcw1
        return (n0, n1)

    zero = jnp.zeros((BP, 1), jnp.float32)
    lax.fori_loop(0, L, step, (zero, zero))
    s = jnp.sum(jnp.sum(ps_ref[...], axis=2), axis=1)
    out_ref[...] = jax.nn.sigmoid(s + fcb_ref[0, 0])[:, None]


def _rnn_head(z, wih0, wih1, whh, bias, fcw0, fcw1, fcb):
    return pl.pallas_call(
        _rnn_body,
        out_shape=jax.ShapeDtypeStruct((L, 1), jnp.float32),
        in_specs=[
            pl.BlockSpec(memory_space=pltpu.VMEM),
            pl.BlockSpec(memory_space=pltpu.VMEM),
            pl.BlockSpec(memory_space=pltpu.VMEM),
            pl.BlockSpec(memory_space=pltpu.SMEM),
            pl.BlockSpec(memory_space=pltpu.SMEM),
            pl.BlockSpec(memory_space=pltpu.VMEM),
            pl.BlockSpec(memory_space=pltpu.VMEM),
            pl.BlockSpec(memory_space=pltpu.SMEM),
        ],
        out_specs=pl.BlockSpec(memory_space=pltpu.VMEM),
        scratch_shapes=[
            pltpu.VMEM((N, 1), jnp.float32),
            pltpu.VMEM((N, 1), jnp.float32),
            pltpu.VMEM((L, BP, 1), jnp.float32),
        ],
    )(z, wih0, wih1, whh, bias, fcw0, fcw1, fcb)


def kernel(x, emb_table, W_ih, W_hh, b_ih, b_hh, fc_W, fc_b):
    z2 = _sc_gather(emb_table.T, x.astype(jnp.int32).T)
    z = jnp.pad(z2[:L * B].reshape(L, B, D),
                ((0, 0), (0, BP - B), (0, 0))).reshape(N, D)

    wih_t = W_ih.astype(jnp.float32).T
    wih0 = wih_t[:, 0:1]
    wih1 = wih_t[:, 1:2]
    bias = (b_ih + b_hh).astype(jnp.float32).reshape(1, H)
    fcw = fc_W.astype(jnp.float32).reshape(B, H)
    fcw0 = jnp.zeros((BP, 1), jnp.float32).at[:B, 0].set(fcw[:, 0])
    fcw1 = jnp.zeros((BP, 1), jnp.float32).at[:B, 0].set(fcw[:, 1])
    whh = W_hh.astype(jnp.float32)
    fcb = fc_b.astype(jnp.float32).reshape(1, 1)

    return _rnn_head(z, wih0, wih1, whh, bias, fcw0, fcw1, fcb)


# speedup vs baseline: 1.0561x; 1.0465x over previous
import functools

import jax
import jax.numpy as jnp
from jax import lax
from jax.experimental import pallas as pl
from jax.experimental.pallas import tpu as pltpu
from jax.experimental.pallas import tpu_sc as plsc

L = 200
B = 10
BP = 16
H = 2
D = 64
N = L * BP
NG = 2048
V8 = 125000


def _sc_gather(ttab, xt):
    info = plsc.get_sparse_core_info()
    nw = info.num_cores * info.num_subcores
    b_per_w = NG // nw
    ck = 8
    mesh = plsc.VectorSubcoreMesh(core_axis_name="c", subcore_axis_name="s")

    nslots = 12

    @functools.partial(
        pl.kernel,
        mesh=mesh,
        out_type=jax.ShapeDtypeStruct((NG, D), jnp.float32),
        compiler_params=pltpu.CompilerParams(needs_layout_passes=False),
        scratch_types=[
            pltpu.VMEM((B, L), jnp.int32),
            pltpu.VMEM((b_per_w,), jnp.int32),
            pltpu.VMEM((D, nslots * 128), jnp.float32),
            pltpu.VMEM((b_per_w, D), jnp.float32),
        ] + [pltpu.SemaphoreType.DMA] * nslots,
    )
    def gather_kernel(ttab_hbm, xt_hbm, out_hbm, xt_v, idx_v, blocks_v,
                      out_v, *sems):
        wid = lax.axis_index("s") * info.num_cores + lax.axis_index("c")
        base = wid * b_per_w
        dvs = [jnp.arange(16, dtype=jnp.int32) + 16 * m
               for m in range(D // 16)]
        pltpu.sync_copy(xt_hbm, xt_v)
        lane = jnp.arange(16, dtype=jnp.int32)
        svals, cvals = [], []
        for jb in range(0, b_per_w, 16):
            n = base + jb + lane
            t = lax.shift_right_logical(n * 6554, 16)
            bb = n - t * 10
            t = jnp.minimum(t, L - 1)
            iv = plsc.load_gather(xt_v, [bb, t])
            for k in range(16):
                e = iv[k]
                svals.append(pl.multiple_of(e & ~jnp.int32(127), 128))
                cvals.append(e & 127)

        def issue(i, slot):
            return pltpu.async_copy(
                ttab_hbm.at[:, pl.ds(svals[i], 128)],
                blocks_v.at[:, pl.ds(128 * slot, 128)], sems[slot])

        def extract(j, slot):
            cf = lax.broadcast(cvals[j] + 128 * slot, (16,))
            jf = jnp.full((16,), j, jnp.int32)
            for dv in dvs:
                vals = plsc.load_gather(blocks_v, [dv, cf])
                plsc.store_scatter(out_v, [jf, dv], vals)

        descs = [None] * nslots
        for i in range(b_per_w + nslots):
            slot = i % nslots
            if i >= nslots:
                descs[slot].wait()
                extract(i - nslots, slot)
            if i < b_per_w:
                descs[slot] = issue(i, slot)
        pltpu.sync_copy(out_v, out_hbm.at[pl.ds(base, b_per_w)])

    return gather_kernel(ttab, xt)


def _rnn_body(z_ref, wih0_ref, wih1_ref, whh_ref, bias_ref, fcw0_ref,
              fcw1_ref, fcb_ref, out_ref, a0_ref, a1_ref, ps_ref):
    a0_ref[...] = (jnp.dot(z_ref[...], wih0_ref[...],
                           preferred_element_type=jnp.float32)
                   + bias_ref[0, 0])
    a1_ref[...] = (jnp.dot(z_ref[...], wih1_ref[...],
                           preferred_element_type=jnp.float32)
                   + bias_ref[0, 1])
    w00 = whh_ref[0, 0]
    w01 = whh_ref[0, 1]
    w10 = whh_ref[1, 0]
    w11 = whh_ref[1, 1]
    fcw0 = fcw0_ref[...]
    fcw1 = fcw1_ref[...]

    def step(t, carry):
        h0, h1 = carry
        a0 = a0_ref[pl.ds(t * B, BP), :]
        a1 = a1_ref[pl.ds(t * B, BP), :]
        n0 = jnp.tanh(a0 + h0 * w00 + h1 * w01)
        n1 = jnp.tanh(a1 + h0 * w10 + h1 * w11)
        ps_ref[t] = n0 * fcw0 + n1 * fcw1
        return (n0, n1)

    zero = jnp.zeros((BP, 1), jnp.float32)
    lax.fori_loop(0, L, step, (zero, zero))
    s = jnp.sum(jnp.sum(ps_ref[...], axis=2), axis=1)
    out_ref[...] = jax.nn.sigmoid(s + fcb_ref[0, 0])[:, None]


def _rnn_head(z, wih0, wih1, whh, bias, fcw0, fcw1, fcb):
    return pl.pallas_call(
        _rnn_body,
        out_shape=jax.ShapeDtypeStruct((L, 1), jnp.float32),
        in_specs=[
            pl.BlockSpec(memory_space=pltpu.VMEM),
            pl.BlockSpec(memory_space=pltpu.VMEM),
            pl.BlockSpec(memory_space=pltpu.VMEM),
            pl.BlockSpec(memory_space=pltpu.SMEM),
            pl.BlockSpec(memory_space=pltpu.SMEM),
            pl.BlockSpec(memory_space=pltpu.VMEM),
            pl.BlockSpec(memory_space=pltpu.VMEM),
            pl.BlockSpec(memory_space=pltpu.SMEM),
        ],
        out_specs=pl.BlockSpec(memory_space=pltpu.VMEM),
        scratch_shapes=[
            pltpu.VMEM((NG, 1), jnp.float32),
            pltpu.VMEM((NG, 1), jnp.float32),
            pltpu.VMEM((L, BP, 1), jnp.float32),
        ],
    )(z, wih0, wih1, whh, bias, fcw0, fcw1, fcb)


def kernel(x, emb_table, W_ih, W_hh, b_ih, b_hh, fc_W, fc_b):
    z = _sc_gather(emb_table.T, x.astype(jnp.int32).T)

    wih_t = W_ih.astype(jnp.float32).T
    wih0 = wih_t[:, 0:1]
    wih1 = wih_t[:, 1:2]
    bias = (b_ih + b_hh).astype(jnp.float32).reshape(1, H)
    fcw = fc_W.astype(jnp.float32).reshape(B, H)
    fcw0 = jnp.zeros((BP, 1), jnp.float32).at[:B, 0].set(fcw[:, 0])
    fcw1 = jnp.zeros((BP, 1), jnp.float32).at[:B, 0].set(fcw[:, 1])
    whh = W_hh.astype(jnp.float32)
    fcb = fc_b.astype(jnp.float32).reshape(1, 1)

    return _rnn_head(z, wih0, wih1, whh, bias, fcw0, fcw1, fcb)


# speedup vs baseline: 1.1260x; 1.0661x over previous
import functools

import jax
import jax.numpy as jnp
from jax import lax
from jax.experimental import pallas as pl
from jax.experimental.pallas import tpu as pltpu
from jax.experimental.pallas import tpu_sc as plsc

L = 200
B = 10
BP = 16
H = 2
D = 64
N = L * BP
NG = 2048
V8 = 125000


def _sc_gather(ttab, xt):
    info = plsc.get_sparse_core_info()
    nw = info.num_cores * info.num_subcores
    b_per_w = NG // nw
    ck = 8
    mesh = plsc.VectorSubcoreMesh(core_axis_name="c", subcore_axis_name="s")

    nslots = 8
    ngroups = b_per_w // nslots

    @functools.partial(
        pl.kernel,
        mesh=mesh,
        out_type=jax.ShapeDtypeStruct((NG, D), jnp.float32),
        compiler_params=pltpu.CompilerParams(needs_layout_passes=False),
        scratch_types=[
            pltpu.VMEM((B, L), jnp.int32),
            pltpu.VMEM((b_per_w + 16, ), jnp.int32),
            pltpu.VMEM((b_per_w + 16, ), jnp.int32),
            pltpu.VMEM((D, nslots * 128), jnp.float32),
            pltpu.VMEM((b_per_w, D), jnp.float32),
        ] + [pltpu.SemaphoreType.DMA] * nslots,
    )
    def gather_kernel(ttab_hbm, xt_hbm, out_hbm, xt_v, sv_v, cv_v, blocks_v,
                      out_v, *sems):
        wid = lax.axis_index("s") * info.num_cores + lax.axis_index("c")
        base = wid * b_per_w
        dvs = [jnp.arange(16, dtype=jnp.int32) + 16 * m
               for m in range(D // 16)]
        pltpu.sync_copy(xt_hbm, xt_v)
        lane = jnp.arange(16, dtype=jnp.int32)
        for jb in range(0, b_per_w, 16):
            n = base + jb + lane
            t = lax.shift_right_logical(n * 6554, 16)
            bb = n - t * 10
            t = jnp.minimum(t, L - 1)
            iv = plsc.load_gather(xt_v, [bb, t])
            sv_v[pl.ds(jb, 16)] = iv & ~jnp.int32(127)
            cv_v[pl.ds(jb, 16)] = iv & 127

        def issue(g, k, svg):
            st = pl.multiple_of(svg[k], 128)
            return pltpu.async_copy(ttab_hbm.at[:, pl.ds(st, 128)],
                                    blocks_v.at[:, pl.ds(128 * k, 128)],
                                    sems[k])

        def wait_extract(g, k, cvg):
            pltpu.make_async_copy(ttab_hbm.at[:, pl.ds(0, 128)],
                                  blocks_v.at[:, pl.ds(128 * k, 128)],
                                  sems[k]).wait()
            cf = lax.broadcast(cvg[k] + 128 * k, (16,))
            jf = lax.broadcast(g * nslots + k, (16,))
            for dv in dvs:
                vals = plsc.load_gather(blocks_v, [dv, cf])
                plsc.store_scatter(out_v, [jf, dv], vals)

        sv0 = sv_v[pl.ds(0, 16)]
        for k in range(nslots):
            issue(0, k, sv0)

        def ring_body(g, carry):
            svp = sv_v[pl.ds((g - 1) * nslots, 16)]
            cvp = cv_v[pl.ds((g - 1) * nslots, 16)]
            svg = sv_v[pl.ds(g * nslots, 16)]
            for k in range(nslots):
                wait_extract(g - 1, k, cvp)
                issue(g, k, svg)
            _ = svp
            return carry

        lax.fori_loop(1, ngroups, ring_body, 0)
        cvl = cv_v[pl.ds((ngroups - 1) * nslots, 16)]
        for k in range(nslots):
            wait_extract(ngroups - 1, k, cvl)
        pltpu.sync_copy(out_v, out_hbm.at[pl.ds(base, b_per_w)])

    return gather_kernel(ttab, xt)


def _rnn_body(z_ref, wih0_ref, wih1_ref, whh_ref, bias_ref, fcw0_ref,
              fcw1_ref, fcb_ref, out_ref, a0_ref, a1_ref, ps_ref):
    a0_ref[...] = (jnp.dot(z_ref[...], wih0_ref[...],
                           preferred_element_type=jnp.float32)
                   + bias_ref[0, 0])
    a1_ref[...] = (jnp.dot(z_ref[...], wih1_ref[...],
                           preferred_element_type=jnp.float32)
                   + bias_ref[0, 1])
    w00 = whh_ref[0, 0]
    w01 = whh_ref[0, 1]
    w10 = whh_ref[1, 0]
    w11 = whh_ref[1, 1]
    fcw0 = fcw0_ref[...]
    fcw1 = fcw1_ref[...]

    def step(t, carry):
        h0, h1 = carry
        a0 = a0_ref[pl.ds(t * B, BP), :]
        a1 = a1_ref[pl.ds(t * B, BP), :]
        n0 = jnp.tanh(a0 + h0 * w00 + h1 * w01)
        n1 = jnp.tanh(a1 + h0 * w10 + h1 * w11)
        ps_ref[t] = n0 * fcw0 + n1 * fcw1
        return (n0, n1)

    zero = jnp.zeros((BP, 1), jnp.float32)
    lax.fori_loop(0, L, step, (zero, zero))
    s = jnp.sum(jnp.sum(ps_ref[...], axis=2), axis=1)
    out_ref[...] = jax.nn.sigmoid(s + fcb_ref[0, 0])[:, None]


def _rnn_head(z, wih0, wih1, whh, bias, fcw0, fcw1, fcb):
    return pl.pallas_call(
        _rnn_body,
        out_shape=jax.ShapeDtypeStruct((L, 1), jnp.float32),
        in_specs=[
            pl.BlockSpec(memory_space=pltpu.VMEM),
            pl.BlockSpec(memory_space=pltpu.VMEM),
            pl.BlockSpec(memory_space=pltpu.VMEM),
            pl.BlockSpec(memory_space=pltpu.SMEM),
            pl.BlockSpec(memory_space=pltpu.SMEM),
            pl.BlockSpec(memory_space=pltpu.VMEM),
            pl.BlockSpec(memory_space=pltpu.VMEM),
            pl.BlockSpec(memory_space=pltpu.SMEM),
        ],
        out_specs=pl.BlockSpec(memory_space=pltpu.VMEM),
        scratch_shapes=[
            pltpu.VMEM((NG, 1), jnp.float32),
            pltpu.VMEM((NG, 1), jnp.float32),
            pltpu.VMEM((L, BP, 1), jnp.float32),
        ],
    )(z, wih0, wih1, whh, bias, fcw0, fcw1, fcb)


def kernel(x, emb_table, W_ih, W_hh, b_ih, b_hh, fc_W, fc_b):
    z = _sc_gather(emb_table.T, x.astype(jnp.int32).T)

    wih_t = W_ih.astype(jnp.float32).T
    wih0 = wih_t[:, 0:1]
    wih1 = wih_t[:, 1:2]
    bias = (b_ih + b_hh).astype(jnp.float32).reshape(1, H)
    fcw = fc_W.astype(jnp.float32).reshape(B, H)
    fcw0 = jnp.zeros((BP, 1), jnp.float32).at[:B, 0].set(fcw[:, 0])
    fcw1 = jnp.zeros((BP, 1), jnp.float32).at[:B, 0].set(fcw[:, 1])
    whh = W_hh.astype(jnp.float32)
    fcb = fc_b.astype(jnp.float32).reshape(1, 1)

    return _rnn_head(z, wih0, wih1, whh, bias, fcw0, fcw1, fcb)
